# Initial kernel scaffold; baseline (speedup 1.0000x reference)
import jax
import jax.numpy as jnp
from jax.experimental import pallas as pl


def kernel(inp_s, inp_t, adj, adj_s, adj_t, W):
    raise NotImplementedError("write your pallas kernel here")


# speedup vs baseline: 1.0540x; 1.0540x over previous
import jax
import jax.numpy as jnp
from jax.experimental import pallas as pl

N = 4096
D = 128
O = 32
BR = 512
NB = N // BR


def _deg_proj_kernel(adj_ref, adjs_ref, adjt_ref, inps_ref, inpt_ref, w_ref,
                     dso_ref, dto_ref, x_ref, y_ref):
    i = pl.program_id(0)
    a = adj_ref[...]
    dso_ref[...] = (jnp.sum(a, axis=1, keepdims=True)
                    + jnp.sum(adjs_ref[...], axis=1, keepdims=True))
    csum = (jnp.sum(a, axis=0, keepdims=True)
            + jnp.sum(adjt_ref[...], axis=0, keepdims=True))

    @pl.when(i == 0)
    def _():
        dto_ref[...] = csum

    @pl.when(i > 0)
    def _():
        dto_ref[...] += csum

    x_ref[...] = jnp.dot(inps_ref[...], w_ref[...],
                         preferred_element_type=jnp.float32)
    y_ref[...] = jnp.dot(inpt_ref[...], w_ref[...],
                         preferred_element_type=jnp.float32)


def _xnew_kernel(adj_ref, adjs_ref, x_ref, y_ref, dso_ref, dtoc_ref,
                 xn_ref, yp_ref):
    i = pl.program_id(0)
    rds_full = jnp.sqrt(dso_ref[...] + 1.0)
    rdt_full = jnp.sqrt(dtoc_ref[...] + 1.0)
    xs = x_ref[...] / rds_full
    yt = y_ref[...] / rdt_full
    acc = (jnp.dot(adjs_ref[...], xs, preferred_element_type=jnp.float32)
           + jnp.dot(adj_ref[...], yt, preferred_element_type=jnp.float32))
    x_blk = x_ref[pl.ds(i * BR, BR), :]
    rds_blk = jnp.sqrt(dso_ref[pl.ds(i * BR, BR), :] + 1.0)
    xn = jnp.maximum((x_blk + acc) / rds_blk, 0.0)
    xn_ref[...] = xn
    contrib = jax.lax.dot_general(adj_ref[...], xn / rds_blk,
                                  (((0,), (0,)), ((), ())),
                                  preferred_element_type=jnp.float32)

    @pl.when(i == 0)
    def _():
        yp_ref[...] = contrib

    @pl.when(i > 0)
    def _():
        yp_ref[...] += contrib


def _ynew_kernel(adjt_ref, y_ref, dtoc_ref, yp_ref, yn_ref):
    i = pl.program_id(0)
    rdt_full = jnp.sqrt(dtoc_ref[...] + 1.0)
    yt = y_ref[...] / rdt_full
    acc = jnp.dot(adjt_ref[...], yt, preferred_element_type=jnp.float32)
    y_blk = y_ref[pl.ds(i * BR, BR), :]
    yp_blk = yp_ref[pl.ds(i * BR, BR), :]
    rdt_blk = jnp.sqrt(dtoc_ref[pl.ds(i * BR, BR), :] + 1.0)
    yn_ref[...] = jnp.maximum((y_blk + acc + yp_blk) / rdt_blk, 0.0)


def kernel(inp_s, inp_t, adj, adj_s, adj_t, W):
    row_blk = pl.BlockSpec((BR, N), lambda i: (i, 0))
    full = lambda shape: pl.BlockSpec(shape, lambda i: (0, 0))

    dso, dto, x, y = pl.pallas_call(
        _deg_proj_kernel,
        grid=(NB,),
        in_specs=[row_blk, row_blk, row_blk,
                  pl.BlockSpec((BR, D), lambda i: (i, 0)),
                  pl.BlockSpec((BR, D), lambda i: (i, 0)),
                  full((D, O))],
        out_specs=[pl.BlockSpec((BR, 1), lambda i: (i, 0)),
                   full((1, N)),
                   pl.BlockSpec((BR, O), lambda i: (i, 0)),
                   pl.BlockSpec((BR, O), lambda i: (i, 0))],
        out_shape=[jax.ShapeDtypeStruct((N, 1), jnp.float32),
                   jax.ShapeDtypeStruct((1, N), jnp.float32),
                   jax.ShapeDtypeStruct((N, O), jnp.float32),
                   jax.ShapeDtypeStruct((N, O), jnp.float32)],
    )(adj, adj_s, adj_t, inp_s, inp_t, W)

    dto_col = dto.reshape(N, 1)

    xn, yp = pl.pallas_call(
        _xnew_kernel,
        grid=(NB,),
        in_specs=[row_blk, row_blk, full((N, O)), full((N, O)),
                  full((N, 1)), full((N, 1))],
        out_specs=[pl.BlockSpec((BR, O), lambda i: (i, 0)),
                   full((N, O))],
        out_shape=[jax.ShapeDtypeStruct((N, O), jnp.float32),
                   jax.ShapeDtypeStruct((N, O), jnp.float32)],
    )(adj, adj_s, x, y, dso, dto_col)

    yn = pl.pallas_call(
        _ynew_kernel,
        grid=(NB,),
        in_specs=[row_blk, full((N, O)), full((N, 1)), full((N, O))],
        out_specs=pl.BlockSpec((BR, O), lambda i: (i, 0)),
        out_shape=jax.ShapeDtypeStruct((N, O), jnp.float32),
    )(adj_t, y, dto_col, yp)

    return (xn, yn)


# speedup vs baseline: 1.0731x; 1.0182x over previous
import jax
import jax.numpy as jnp
from jax.experimental import pallas as pl

N = 4096
D = 128
O = 32
BR = 512
NB = N // BR


def _deg_proj_kernel(adj_ref, adjs_ref, adjt_ref, inps_ref, inpt_ref, w_ref,
                     dso_ref, dto_ref, x_ref, y_ref):
    i = pl.program_id(0)
    a = adj_ref[...]
    dso_ref[...] = (jnp.sum(a, axis=1, keepdims=True)
                    + jnp.sum(adjs_ref[...], axis=1, keepdims=True))
    csum = (jnp.sum(a, axis=0, keepdims=True)
            + jnp.sum(adjt_ref[...], axis=0, keepdims=True))

    @pl.when(i == 0)
    def _():
        dto_ref[...] = csum

    @pl.when(i > 0)
    def _():
        dto_ref[...] += csum

    x_ref[...] = jnp.dot(inps_ref[...], w_ref[...],
                         preferred_element_type=jnp.float32)
    y_ref[...] = jnp.dot(inpt_ref[...], w_ref[...],
                         preferred_element_type=jnp.float32)


def _xnew_kernel(adj_ref, adjs_ref, x_ref, y_ref, dso_ref, dtoc_ref,
                 xn_ref, yp_ref):
    i = pl.program_id(0)
    rds_full = jnp.sqrt(dso_ref[...] + 1.0)
    rdt_full = jnp.sqrt(dtoc_ref[...] + 1.0)
    xs = (x_ref[...] / rds_full).astype(jnp.bfloat16)
    yt = (y_ref[...] / rdt_full).astype(jnp.bfloat16)
    a = adj_ref[...].astype(jnp.bfloat16)
    acc = (jnp.dot(adjs_ref[...].astype(jnp.bfloat16), xs,
                   preferred_element_type=jnp.float32)
           + jnp.dot(a, yt, preferred_element_type=jnp.float32))
    x_blk = x_ref[pl.ds(i * BR, BR), :]
    rds_blk = jnp.sqrt(dso_ref[pl.ds(i * BR, BR), :] + 1.0)
    xn = jnp.maximum((x_blk + acc) / rds_blk, 0.0)
    xn_ref[...] = xn
    contrib = jax.lax.dot_general(a, (xn / rds_blk).astype(jnp.bfloat16),
                                  (((0,), (0,)), ((), ())),
                                  preferred_element_type=jnp.float32)

    @pl.when(i == 0)
    def _():
        yp_ref[...] = contrib

    @pl.when(i > 0)
    def _():
        yp_ref[...] += contrib


def _ynew_kernel(adjt_ref, y_ref, dtoc_ref, yp_ref, yn_ref):
    i = pl.program_id(0)
    rdt_full = jnp.sqrt(dtoc_ref[...] + 1.0)
    yt = (y_ref[...] / rdt_full).astype(jnp.bfloat16)
    acc = jnp.dot(adjt_ref[...].astype(jnp.bfloat16), yt,
                  preferred_element_type=jnp.float32)
    y_blk = y_ref[pl.ds(i * BR, BR), :]
    yp_blk = yp_ref[pl.ds(i * BR, BR), :]
    rdt_blk = jnp.sqrt(dtoc_ref[pl.ds(i * BR, BR), :] + 1.0)
    yn_ref[...] = jnp.maximum((y_blk + acc + yp_blk) / rdt_blk, 0.0)


def kernel(inp_s, inp_t, adj, adj_s, adj_t, W):
    row_blk = pl.BlockSpec((BR, N), lambda i: (i, 0))
    full = lambda shape: pl.BlockSpec(shape, lambda i: (0, 0))

    dso, dto, x, y = pl.pallas_call(
        _deg_proj_kernel,
        grid=(NB,),
        in_specs=[row_blk, row_blk, row_blk,
                  pl.BlockSpec((BR, D), lambda i: (i, 0)),
                  pl.BlockSpec((BR, D), lambda i: (i, 0)),
                  full((D, O))],
        out_specs=[pl.BlockSpec((BR, 1), lambda i: (i, 0)),
                   full((1, N)),
                   pl.BlockSpec((BR, O), lambda i: (i, 0)),
                   pl.BlockSpec((BR, O), lambda i: (i, 0))],
        out_shape=[jax.ShapeDtypeStruct((N, 1), jnp.float32),
                   jax.ShapeDtypeStruct((1, N), jnp.float32),
                   jax.ShapeDtypeStruct((N, O), jnp.float32),
                   jax.ShapeDtypeStruct((N, O), jnp.float32)],
    )(adj, adj_s, adj_t, inp_s, inp_t, W)

    dto_col = dto.reshape(N, 1)

    xn, yp = pl.pallas_call(
        _xnew_kernel,
        grid=(NB,),
        in_specs=[row_blk, row_blk, full((N, O)), full((N, O)),
                  full((N, 1)), full((N, 1))],
        out_specs=[pl.BlockSpec((BR, O), lambda i: (i, 0)),
                   full((N, O))],
        out_shape=[jax.ShapeDtypeStruct((N, O), jnp.float32),
                   jax.ShapeDtypeStruct((N, O), jnp.float32)],
    )(adj, adj_s, x, y, dso, dto_col)

    yn = pl.pallas_call(
        _ynew_kernel,
        grid=(NB,),
        in_specs=[row_blk, full((N, O)), full((N, 1)), full((N, O))],
        out_specs=pl.BlockSpec((BR, O), lambda i: (i, 0)),
        out_shape=jax.ShapeDtypeStruct((N, O), jnp.float32),
    )(adj_t, y, dto_col, yp)

    return (xn, yn)
